# baseline (device time: 10802 ns/iter reference)
import jax
import jax.numpy as jnp
from jax import lax
from jax.experimental import pallas as pl
from jax.experimental.pallas import tpu as pltpu

K = 8
LANES = 128

SORT8_NET = (
    (0, 1), (2, 3), (4, 5), (6, 7),
    (0, 2), (1, 3), (4, 6), (5, 7),
    (1, 2), (5, 6),
    (0, 4), (1, 5), (2, 6), (3, 7),
    (2, 4), (3, 5),
    (1, 2), (3, 4), (5, 6),
)


def kernel(x):
    m, n = x.shape
    dtype = x.dtype
    blocks = n // LANES
    assert blocks == K, "merge tree below assumes 8 blocks of 128 lanes"

    def _bitonic8_desc(L):
        for d in (4, 2, 1):
            L = [
                jnp.minimum(L[i], L[i ^ d]) if i & d
                else jnp.maximum(L[i], L[i ^ d])
                for i in range(K)
            ]
        return L

    def _top8_desc_asc(vals):
        S = [vals[:, i * LANES : (i + 1) * LANES] for i in range(blocks)]
        for a, b in SORT8_NET:
            hi = jnp.maximum(S[a], S[b])
            lo = jnp.minimum(S[a], S[b])
            S[a], S[b] = hi, lo
        for step in range(7):
            d = 1 << step
            T = [pltpu.roll(s, LANES - d, 1) for s in S]
            L = [jnp.maximum(S[i], T[K - 1 - i]) for i in range(K)]
            S = _bitonic8_desc(L)
        out_iota = lax.broadcasted_iota(jnp.int32, (m, K), 1)
        desc = jnp.full((m, K), float("-inf"), dtype)
        asc = jnp.full((m, K), float("-inf"), dtype)
        for i in range(K):
            col = S[i][:, 0:1]
            desc = jnp.where(out_iota == i, col, desc)
            asc = jnp.where(out_iota == K - 1 - i, col, asc)
        return desc, asc

    def _merge_top8(desc_mine, asc_theirs):
        out = jnp.maximum(desc_mine, asc_theirs)
        iota8 = lax.broadcasted_iota(jnp.int32, (m, K), 1)
        for d in (4, 2, 1):
            up = pltpu.roll(out, d, 1)
            down = pltpu.roll(out, K - d, 1)
            hi_lane = (iota8 & d) != 0
            partner = jnp.where(hi_lane, up, down)
            out = jnp.where(
                hi_lane, jnp.minimum(out, partner), jnp.maximum(out, partner)
            )
        return out

    def body(x_ref, out_ref, send_ref, recv_ref, send_sem, recv_sem):
        my_x = lax.axis_index("x")
        my_y = lax.axis_index("y")
        nbr = (my_x, 1 - my_y)

        barrier_sem = pltpu.get_barrier_semaphore()
        pl.semaphore_signal(
            barrier_sem, inc=1, device_id=nbr,
            device_id_type=pl.DeviceIdType.MESH,
        )

        desc, asc = _top8_desc_asc(x_ref[:, :])
        send_ref[:, :] = asc

        pl.semaphore_wait(barrier_sem, 1)

        rdma = pltpu.make_async_remote_copy(
            src_ref=send_ref,
            dst_ref=recv_ref,
            send_sem=send_sem,
            recv_sem=recv_sem,
            device_id=nbr,
            device_id_type=pl.DeviceIdType.MESH,
        )
        rdma.start()
        rdma.wait()

        out_ref[:, :] = _merge_top8(desc, recv_ref[:, :])

    return pl.pallas_call(
        body,
        out_shape=jax.ShapeDtypeStruct((m, K), dtype),
        in_specs=[pl.BlockSpec(memory_space=pltpu.VMEM)],
        out_specs=pl.BlockSpec(memory_space=pltpu.VMEM),
        scratch_shapes=[
            pltpu.VMEM((m, K), dtype),
            pltpu.VMEM((m, K), dtype),
            pltpu.SemaphoreType.DMA,
            pltpu.SemaphoreType.DMA,
        ],
        compiler_params=pltpu.CompilerParams(collective_id=0),
    )(x)


# device time: 9373 ns/iter; 1.1525x vs baseline; 1.1525x over previous
import jax
import jax.numpy as jnp
from jax import lax
from jax.experimental import pallas as pl
from jax.experimental.pallas import tpu as pltpu

K = 8
LANES = 128
IDX_MASK = 0x3FF
KEY_MIN = -(2**31)

SORT8_NET = (
    (0, 1), (2, 3), (4, 5), (6, 7),
    (0, 2), (1, 3), (4, 6), (5, 7),
    (1, 2), (5, 6),
    (0, 4), (1, 5), (2, 6), (3, 7),
    (2, 4), (3, 5),
    (1, 2), (3, 4), (5, 6),
)


def kernel(x):
    m, n = x.shape
    dtype = x.dtype
    blocks = n // LANES

    def _pack(vals):
        b = lax.bitcast_convert_type(vals, jnp.int32)
        s = jnp.where(b >= 0, b, b ^ 0x7FFFFFFF)
        iota = lax.broadcasted_iota(jnp.int32, (m, n), 1)
        return (s & ~IDX_MASK) | iota

    def _unpack(keys):
        s = keys & ~IDX_MASK
        b = jnp.where(s >= 0, s, s ^ 0x7FFFFFFF)
        return lax.bitcast_convert_type(b, dtype)

    def _top8_desc_asc(keys):
        S = [keys[:, i * LANES : (i + 1) * LANES] for i in range(blocks)]
        for a, b in SORT8_NET:
            hi = jnp.maximum(S[a], S[b])
            lo = jnp.minimum(S[a], S[b])
            S[a], S[b] = hi, lo
        out_iota = lax.broadcasted_iota(jnp.int32, (m, K), 1)
        desc = jnp.full((m, K), KEY_MIN, jnp.int32)
        asc = jnp.full((m, K), KEY_MIN, jnp.int32)
        for k in range(K):
            mx = jnp.max(S[0], axis=1, keepdims=True)
            desc = jnp.where(out_iota == k, mx, desc)
            asc = jnp.where(out_iota == K - 1 - k, mx, asc)
            mask = S[0] == mx
            for i in range(blocks - 1):
                S[i] = jnp.where(mask, S[i + 1], S[i])
            S[blocks - 1] = jnp.where(mask, KEY_MIN, S[blocks - 1])
        return desc, asc

    def _merge_top8(desc_mine, asc_theirs):
        out = jnp.maximum(desc_mine, asc_theirs)
        iota8 = lax.broadcasted_iota(jnp.int32, (m, K), 1)
        for d in (4, 2, 1):
            up = pltpu.roll(out, d, 1)
            down = pltpu.roll(out, K - d, 1)
            hi_lane = (iota8 & d) != 0
            partner = jnp.where(hi_lane, up, down)
            out = jnp.where(
                hi_lane, jnp.minimum(out, partner), jnp.maximum(out, partner)
            )
        return out

    def body(x_ref, out_ref, send_ref, recv_ref, send_sem, recv_sem):
        my_x = lax.axis_index("x")
        my_y = lax.axis_index("y")
        nbr = (my_x, 1 - my_y)

        barrier_sem = pltpu.get_barrier_semaphore()
        pl.semaphore_signal(
            barrier_sem, inc=1, device_id=nbr,
            device_id_type=pl.DeviceIdType.MESH,
        )

        desc, asc = _top8_desc_asc(_pack(x_ref[:, :]))
        send_ref[:, :] = asc

        pl.semaphore_wait(barrier_sem, 1)

        rdma = pltpu.make_async_remote_copy(
            src_ref=send_ref,
            dst_ref=recv_ref,
            send_sem=send_sem,
            recv_sem=recv_sem,
            device_id=nbr,
            device_id_type=pl.DeviceIdType.MESH,
        )
        rdma.start()
        rdma.wait()

        out_ref[:, :] = _unpack(_merge_top8(desc, recv_ref[:, :]))

    return pl.pallas_call(
        body,
        out_shape=jax.ShapeDtypeStruct((m, K), dtype),
        in_specs=[pl.BlockSpec(memory_space=pltpu.VMEM)],
        out_specs=pl.BlockSpec(memory_space=pltpu.VMEM),
        scratch_shapes=[
            pltpu.VMEM((m, K), jnp.int32),
            pltpu.VMEM((m, K), jnp.int32),
            pltpu.SemaphoreType.DMA,
            pltpu.SemaphoreType.DMA,
        ],
        compiler_params=pltpu.CompilerParams(collective_id=0),
    )(x)
